# baseline (speedup 1.0000x reference)
import functools

import jax
import jax.numpy as jnp
import numpy as np
from jax import lax
from jax.experimental import pallas as pl
from jax.experimental.pallas import tpu as pltpu
from jax.experimental.pallas import tpu_sc as plsc

N = 10000
E = 160000
D = 128
R = 4
NV = 3

NW = 32
CHUNK = 128
NCHUNK = 40
EPW = NCHUNK * CHUNK
E_PAD = NW * EPW
ROWS_PT = 1000

BN = 1000


_sc_mesh = plsc.VectorSubcoreMesh(core_axis_name="c", subcore_axis_name="s")


@functools.partial(
    pl.kernel,
    out_type=jax.ShapeDtypeStruct((2, N, D), jnp.float32),
    mesh=_sc_mesh,
    compiler_params=pltpu.CompilerParams(use_tc_tiling_on_sc=False),
    scratch_types=[
        pltpu.VMEM((NCHUNK, CHUNK), jnp.int32),
        pltpu.VMEM((NCHUNK, CHUNK), jnp.int32),
        pltpu.VMEM((NCHUNK, CHUNK), jnp.float32),
        [pltpu.VMEM((CHUNK, D // 2), jnp.int32)] * 2,
        pltpu.VMEM((CHUNK, D), jnp.float32),
        pltpu.VMEM_SHARED((N, D), jnp.float32),
        [pltpu.SemaphoreType.DMA] * 4,
    ],
)
def _edge_pass(table, gidx, dst, w, zeros, out, gidx_v, dst_v, w_v, rows,
               scaled, acc, sems):
    cid = lax.axis_index("c")
    sid = lax.axis_index("s")
    wid = cid * 16 + sid

    off = pl.multiple_of(sid * ROWS_PT, 8)

    @pl.when(sid < N // ROWS_PT)
    def _init():
        pltpu.sync_copy(zeros.at[pl.ds(off, ROWS_PT)],
                        acc.at[pl.ds(off, ROWS_PT)])
    pltpu.sync_copy(gidx.at[wid], gidx_v)
    pltpu.sync_copy(dst.at[wid], dst_v)
    pltpu.sync_copy(w.at[wid], w_v)
    plsc.subcore_barrier()

    def scale_scatter(buf, c):
        def grp_body(g, carry2):
            wrow = w_v[c, pl.ds(g * 16, 16)]
            for k in range(16):
                wvec = jnp.full((16,), wrow[k], dtype=jnp.float32)
                i = g * 16 + k
                for l in range(4):
                    v = buf[i, pl.ds(16 * l, 16)]
                    e = lax.bitcast_convert_type(v << 16, jnp.float32)
                    o = lax.bitcast_convert_type(v & jnp.int32(-65536),
                                                 jnp.float32)
                    scaled[i, pl.ds(32 * l, 16)] = e * wvec
                    scaled[i, pl.ds(32 * l + 16, 16)] = o * wvec
            return carry2

        lax.fori_loop(0, CHUNK // 16, grp_body, 0)
        pltpu.sync_copy(scaled, acc.at[dst_v.at[c]], add=True)

    def issue(c, b):
        pltpu.async_copy(table.at[gidx_v.at[c]], rows[b], sems[b])

    def drain(c, b):
        pltpu.make_async_copy(table.at[gidx_v.at[c]], rows[b], sems[b]).wait()

    NB = 2
    for b in range(NB):
        issue(b, b)

    def ring_body(t, carry):
        for b in range(NB):
            c = t * NB + b
            drain(c, b)
            scale_scatter(rows[b], c)
            issue(c + NB, b)
        return carry

    lax.fori_loop(0, NCHUNK // NB - 1, ring_body, 0)
    for b in range(NB):
        c = NCHUNK - NB + b
        drain(c, b)
        scale_scatter(rows[b], c)
    plsc.subcore_barrier()

    @pl.when(sid < N // ROWS_PT)
    def _writeback():
        pltpu.sync_copy(acc.at[pl.ds(off, ROWS_PT)],
                        out.at[cid, pl.ds(off, ROWS_PT)])


def _mm_body(x_ref, w_ref, o_ref):
    o_ref[0] = jnp.dot(
        x_ref[...], w_ref[0], preferred_element_type=jnp.float32
    ).astype(jnp.bfloat16)


def _mm_batched(x, w):
    k = w.shape[0]
    return pl.pallas_call(
        _mm_body,
        grid=(N // BN, k),
        in_specs=[
            pl.BlockSpec((BN, D), lambda j, r: (j, 0)),
            pl.BlockSpec((1, D, D), lambda j, r: (r, 0, 0)),
        ],
        out_specs=pl.BlockSpec((1, BN, D), lambda j, r: (r, j, 0)),
        out_shape=jax.ShapeDtypeStruct((k, N, D), jnp.bfloat16),
    )(x, w)


def _relu_mm_body(p_ref, w_ref, o_ref):
    f = jnp.maximum(p_ref[0] + p_ref[1], 0.0)
    o_ref[0] = jnp.dot(
        f, w_ref[0], preferred_element_type=jnp.float32
    ).astype(jnp.bfloat16)


def _relu_mm(p, w):
    return pl.pallas_call(
        _relu_mm_body,
        grid=(N // BN, R),
        in_specs=[
            pl.BlockSpec((2, BN, D), lambda j, r: (0, j, 0)),
            pl.BlockSpec((1, D, D), lambda j, r: (r, 0, 0)),
        ],
        out_specs=pl.BlockSpec((1, BN, D), lambda j, r: (r, j, 0)),
        out_shape=jax.ShapeDtypeStruct((R, N, D), jnp.bfloat16),
    )(p, w)


def _att_body(q1_ref, q2_ref, q3_ref, w1_ref, b1_ref, w2_ref, o_ref):
    e1 = q1_ref[0] + q1_ref[1]
    e2 = q2_ref[0] + q2_ref[1]
    e3 = q3_ref[0] + q3_ref[1]
    w1 = w1_ref[...]
    b1 = b1_ref[...]
    w2 = w2_ref[...]

    def score(e):
        h = jnp.tanh(jnp.dot(e, w1, preferred_element_type=jnp.float32) + b1)
        return jnp.sum(h * w2, axis=1, keepdims=True)

    s1, s2, s3 = score(e1), score(e2), score(e3)
    m = jnp.maximum(jnp.maximum(s1, s2), s3)
    a1 = jnp.exp(s1 - m)
    a2 = jnp.exp(s2 - m)
    a3 = jnp.exp(s3 - m)
    denom = a1 + a2 + a3
    o_ref[...] = (a1 * e1 + a2 * e2 + a3 * e3) / denom


def _attention(q1, q2, q3, att_w1, att_b1, att_w2):
    b1 = att_b1.reshape(1, D)
    w2 = att_w2.reshape(1, D)
    return pl.pallas_call(
        _att_body,
        grid=(N // BN,),
        in_specs=[
            pl.BlockSpec((2, BN, D), lambda j: (0, j, 0)),
            pl.BlockSpec((2, BN, D), lambda j: (0, j, 0)),
            pl.BlockSpec((2, BN, D), lambda j: (0, j, 0)),
            pl.BlockSpec((D, D), lambda j: (0, 0)),
            pl.BlockSpec((1, D), lambda j: (0, 0)),
            pl.BlockSpec((1, D), lambda j: (0, 0)),
        ],
        out_specs=pl.BlockSpec((BN, D), lambda j: (j, 0)),
        out_shape=jax.ShapeDtypeStruct((N, D), jnp.float32),
    )(q1, q2, q3, att_w1, b1, w2)


def _edge_arrays(edge_index, rel_type, edge_weight):
    src = edge_index[0]
    dst = edge_index[1]
    gidx = rel_type * N + src
    pad = E_PAD - E
    gidx = jnp.pad(gidx, (0, pad)).reshape(NW, NCHUNK, CHUNK)
    dst = jnp.pad(dst, (0, pad)).reshape(NW, NCHUNK, CHUNK)
    w = jnp.pad(edge_weight, (0, pad)).reshape(NW, NCHUNK, CHUNK)
    return gidx, dst, w


def kernel(node_features, edge_index1, rel_type1, edge_weight1, edge_index2,
           rel_type2, edge_weight2, edge_index3, rel_type3, edge_weight3,
           rel_emb_v1_l1, rel_emb_v1_l2, rel_emb_v2_l1, rel_emb_v2_l2,
           rel_emb_v3_l1, rel_emb_v3_l2, att_w1, att_b1, att_w2, att_b2):
    del att_b2
    zeros = jnp.zeros((N, D), jnp.float32)
    edges = [
        _edge_arrays(edge_index1, rel_type1, edge_weight1),
        _edge_arrays(edge_index2, rel_type2, edge_weight2),
        _edge_arrays(edge_index3, rel_type3, edge_weight3),
    ]
    perm = np.empty(D, np.int32)
    for g in range(D // 32):
        for k in range(16):
            perm[32 * g + 2 * k] = 32 * g + k
            perm[32 * g + 2 * k + 1] = 32 * g + 16 + k
    w_l1 = jnp.concatenate(
        [rel_emb_v1_l1, rel_emb_v2_l1, rel_emb_v3_l1], 0)[:, :, perm]
    w_l2 = [w[:, :, perm] for w in
            (rel_emb_v1_l2, rel_emb_v2_l2, rel_emb_v3_l2)]

    def as_words(y):
        return lax.bitcast_convert_type(
            y.reshape(R * N, D // 2, 2), jnp.int32)

    y1 = _mm_batched(node_features, w_l1)

    p2 = []
    for v in range(NV):
        gidx, dst, w = edges[v]
        table1 = as_words(y1[R * v:R * v + R])
        p1 = _edge_pass(table1, gidx, dst, w, zeros)
        y2 = as_words(_relu_mm(p1, w_l2[v]))
        p2.append(_edge_pass(y2, gidx, dst, w, zeros))

    return _attention(p2[0], p2[1], p2[2], att_w1, att_b1, att_w2)


# speedup vs baseline: 3.3991x; 3.3991x over previous
import functools

import jax
import jax.numpy as jnp
import numpy as np
from jax import lax
from jax.experimental import pallas as pl
from jax.experimental.pallas import tpu as pltpu
from jax.experimental.pallas import tpu_sc as plsc

N = 10000
E = 160000
D = 128
R = 4
NV = 3

NW = 32
CHUNK = 128
NCHUNK = 40
EPW = NCHUNK * CHUNK
E_PAD = NW * EPW
ROWS_PT = 1000

BN = 1000


_sc_mesh = plsc.VectorSubcoreMesh(core_axis_name="c", subcore_axis_name="s")


@functools.partial(
    pl.kernel,
    out_type=jax.ShapeDtypeStruct((2, N, D), jnp.float32),
    mesh=_sc_mesh,
    scratch_types=[
        pltpu.VMEM((NCHUNK, CHUNK), jnp.int32),
        pltpu.VMEM((NCHUNK, CHUNK), jnp.int32),
        pltpu.VMEM((NCHUNK, CHUNK), jnp.float32),
        [pltpu.VMEM((CHUNK, D), jnp.float32)] * 2,
        pltpu.VMEM_SHARED((N, D), jnp.float32),
        [pltpu.SemaphoreType.DMA] * 4,
    ],
)
def _edge_pass(table, gidx, dst, w, zeros, out, gidx_v, dst_v, w_v, rows,
               acc, sems):
    cid = lax.axis_index("c")
    sid = lax.axis_index("s")
    wid = cid * 16 + sid

    off = pl.multiple_of(sid * ROWS_PT, 8)

    @pl.when(sid < N // ROWS_PT)
    def _init():
        pltpu.sync_copy(zeros.at[pl.ds(off, ROWS_PT)],
                        acc.at[pl.ds(off, ROWS_PT)])
    pltpu.sync_copy(gidx.at[wid], gidx_v)
    pltpu.sync_copy(dst.at[wid], dst_v)
    pltpu.sync_copy(w.at[wid], w_v)
    plsc.subcore_barrier()

    def scale_scatter(buf, c):
        def grp_body(g, carry2):
            wrow = w_v[c, pl.ds(g * 16, 16)]
            for k in range(16):
                wvec = jnp.full((16,), wrow[k], dtype=jnp.float32)
                i = g * 16 + k
                for l in range(8):
                    sl = pl.ds(l * 16, 16)
                    buf[i, sl] = buf[i, sl] * wvec
            return carry2

        lax.fori_loop(0, CHUNK // 16, grp_body, 0)
        pltpu.sync_copy(buf, acc.at[dst_v.at[c]], add=True)

    def issue(c, b):
        pltpu.async_copy(table.at[gidx_v.at[c]], rows[b], sems[b])

    def drain(c, b):
        pltpu.make_async_copy(table.at[gidx_v.at[c]], rows[b], sems[b]).wait()

    NB = 2
    for b in range(NB):
        issue(b, b)

    def ring_body(t, carry):
        for b in range(NB):
            c = t * NB + b
            drain(c, b)
            scale_scatter(rows[b], c)
            issue(c + NB, b)
        return carry

    lax.fori_loop(0, NCHUNK // NB - 1, ring_body, 0)
    for b in range(NB):
        c = NCHUNK - NB + b
        drain(c, b)
        scale_scatter(rows[b], c)
    plsc.subcore_barrier()

    @pl.when(sid < N // ROWS_PT)
    def _writeback():
        pltpu.sync_copy(acc.at[pl.ds(off, ROWS_PT)],
                        out.at[cid, pl.ds(off, ROWS_PT)])


def _mm_body(x_ref, w_ref, o_ref):
    o_ref[0] = jnp.dot(x_ref[...], w_ref[0], preferred_element_type=jnp.float32)


def _mm_batched(x, w):
    k = w.shape[0]
    return pl.pallas_call(
        _mm_body,
        grid=(N // BN, k),
        in_specs=[
            pl.BlockSpec((BN, D), lambda j, r: (j, 0)),
            pl.BlockSpec((1, D, D), lambda j, r: (r, 0, 0)),
        ],
        out_specs=pl.BlockSpec((1, BN, D), lambda j, r: (r, j, 0)),
        out_shape=jax.ShapeDtypeStruct((k, N, D), jnp.float32),
    )(x, w)


def _relu_mm_body(p_ref, w_ref, o_ref):
    f = jnp.maximum(p_ref[0] + p_ref[1], 0.0)
    o_ref[0] = jnp.dot(f, w_ref[0], preferred_element_type=jnp.float32)


def _relu_mm(p, w):
    return pl.pallas_call(
        _relu_mm_body,
        grid=(N // BN, R),
        in_specs=[
            pl.BlockSpec((2, BN, D), lambda j, r: (0, j, 0)),
            pl.BlockSpec((1, D, D), lambda j, r: (r, 0, 0)),
        ],
        out_specs=pl.BlockSpec((1, BN, D), lambda j, r: (r, j, 0)),
        out_shape=jax.ShapeDtypeStruct((R, N, D), jnp.float32),
    )(p, w)


def _att_body(q1_ref, q2_ref, q3_ref, w1_ref, b1_ref, w2_ref, o_ref):
    e1 = q1_ref[0] + q1_ref[1]
    e2 = q2_ref[0] + q2_ref[1]
    e3 = q3_ref[0] + q3_ref[1]
    w1 = w1_ref[...]
    b1 = b1_ref[...]
    w2 = w2_ref[...]

    def score(e):
        h = jnp.tanh(jnp.dot(e, w1, preferred_element_type=jnp.float32) + b1)
        return jnp.sum(h * w2, axis=1, keepdims=True)

    s1, s2, s3 = score(e1), score(e2), score(e3)
    m = jnp.maximum(jnp.maximum(s1, s2), s3)
    a1 = jnp.exp(s1 - m)
    a2 = jnp.exp(s2 - m)
    a3 = jnp.exp(s3 - m)
    denom = a1 + a2 + a3
    o_ref[...] = (a1 * e1 + a2 * e2 + a3 * e3) / denom


def _attention(q1, q2, q3, att_w1, att_b1, att_w2):
    b1 = att_b1.reshape(1, D)
    w2 = att_w2.reshape(1, D)
    return pl.pallas_call(
        _att_body,
        grid=(N // BN,),
        in_specs=[
            pl.BlockSpec((2, BN, D), lambda j: (0, j, 0)),
            pl.BlockSpec((2, BN, D), lambda j: (0, j, 0)),
            pl.BlockSpec((2, BN, D), lambda j: (0, j, 0)),
            pl.BlockSpec((D, D), lambda j: (0, 0)),
            pl.BlockSpec((1, D), lambda j: (0, 0)),
            pl.BlockSpec((1, D), lambda j: (0, 0)),
        ],
        out_specs=pl.BlockSpec((BN, D), lambda j: (j, 0)),
        out_shape=jax.ShapeDtypeStruct((N, D), jnp.float32),
    )(q1, q2, q3, att_w1, b1, w2)


def _edge_arrays(edge_index, rel_type, edge_weight):
    src = edge_index[0]
    dst = edge_index[1]
    gidx = rel_type * N + src
    pad = E_PAD - E
    pad_idx = jnp.arange(pad, dtype=jnp.int32) * 8 % (R * N)
    gidx = jnp.concatenate([gidx, pad_idx]).reshape(NW, NCHUNK, CHUNK)
    dst = jnp.pad(dst, (0, pad)).reshape(NW, NCHUNK, CHUNK)
    w = jnp.pad(edge_weight, (0, pad)).reshape(NW, NCHUNK, CHUNK)
    return gidx, dst, w


def kernel(node_features, edge_index1, rel_type1, edge_weight1, edge_index2,
           rel_type2, edge_weight2, edge_index3, rel_type3, edge_weight3,
           rel_emb_v1_l1, rel_emb_v1_l2, rel_emb_v2_l1, rel_emb_v2_l2,
           rel_emb_v3_l1, rel_emb_v3_l2, att_w1, att_b1, att_w2, att_b2):
    del att_b2
    zeros = jnp.zeros((N, D), jnp.float32)
    edges = [
        _edge_arrays(edge_index1, rel_type1, edge_weight1),
        _edge_arrays(edge_index2, rel_type2, edge_weight2),
        _edge_arrays(edge_index3, rel_type3, edge_weight3),
    ]
    w_l1 = jnp.concatenate([rel_emb_v1_l1, rel_emb_v2_l1, rel_emb_v3_l1], 0)
    w_l2 = [rel_emb_v1_l2, rel_emb_v2_l2, rel_emb_v3_l2]

    y1 = _mm_batched(node_features, w_l1)

    p2 = []
    for v in range(NV):
        gidx, dst, w = edges[v]
        table1 = y1[R * v:R * v + R].reshape(R * N, D)
        p1 = _edge_pass(table1, gidx, dst, w, zeros)
        y2 = _relu_mm(p1, w_l2[v]).reshape(R * N, D)
        p2.append(_edge_pass(y2, gidx, dst, w, zeros))

    return _attention(p2[0], p2[1], p2[2], att_w1, att_b1, att_w2)


# speedup vs baseline: 3.5653x; 1.0489x over previous
import functools

import jax
import jax.numpy as jnp
import numpy as np
from jax import lax
from jax.experimental import pallas as pl
from jax.experimental.pallas import tpu as pltpu
from jax.experimental.pallas import tpu_sc as plsc

N = 10000
E = 160000
D = 128
R = 4
NV = 3

NW = 32
CHUNK = 128
NCHUNK = 40
EPW = NCHUNK * CHUNK
E_PAD = NW * EPW
ROWS_PT = 1000

BN = 1000


_sc_mesh = plsc.VectorSubcoreMesh(core_axis_name="c", subcore_axis_name="s")


@functools.partial(
    pl.kernel,
    out_type=jax.ShapeDtypeStruct((2, N, D), jnp.float32),
    mesh=_sc_mesh,
    scratch_types=[
        pltpu.VMEM((NCHUNK, CHUNK), jnp.int32),
        pltpu.VMEM((NCHUNK, CHUNK), jnp.int32),
        pltpu.VMEM((NCHUNK, CHUNK), jnp.float32),
        [pltpu.VMEM((CHUNK, D), jnp.float32)] * 2,
        pltpu.VMEM_SHARED((N, D), jnp.float32),
        [pltpu.SemaphoreType.DMA] * 4,
    ],
)
def _edge_pass(table, gidx, dst, w, zeros, out, gidx_v, dst_v, w_v, rows,
               acc, sems):
    cid = lax.axis_index("c")
    sid = lax.axis_index("s")
    wid = cid * 16 + sid

    off = pl.multiple_of(sid * ROWS_PT, 8)

    @pl.when(sid < N // ROWS_PT)
    def _init():
        pltpu.sync_copy(zeros.at[pl.ds(off, ROWS_PT)],
                        acc.at[pl.ds(off, ROWS_PT)])
    pltpu.sync_copy(gidx.at[wid], gidx_v)
    pltpu.sync_copy(dst.at[wid], dst_v)
    pltpu.sync_copy(w.at[wid], w_v)
    plsc.subcore_barrier()

    def scale_scatter(buf, c):
        def grp_body(g, carry2):
            wrow = w_v[c, pl.ds(g * 16, 16)]
            for k in range(16):
                wvec = jnp.full((16,), wrow[k], dtype=jnp.float32)
                i = g * 16 + k
                for l in range(8):
                    sl = pl.ds(l * 16, 16)
                    buf[i, sl] = buf[i, sl] * wvec
            return carry2

        lax.fori_loop(0, CHUNK // 16, grp_body, 0)
        pltpu.sync_copy(buf, acc.at[dst_v.at[c]], add=True)

    def issue(c, b):
        pltpu.async_copy(table.at[gidx_v.at[c]], rows[b], sems[b])

    def drain(c, b):
        pltpu.make_async_copy(table.at[gidx_v.at[c]], rows[b], sems[b]).wait()

    NB = 2
    for b in range(NB):
        issue(b, b)

    def ring_body(t, carry):
        for b in range(NB):
            c = t * NB + b
            drain(c, b)
            scale_scatter(rows[b], c)
            issue(c + NB, b)
        return carry

    lax.fori_loop(0, NCHUNK // NB - 1, ring_body, 0)
    for b in range(NB):
        c = NCHUNK - NB + b
        drain(c, b)
        scale_scatter(rows[b], c)
    plsc.subcore_barrier()

    @pl.when(sid < N // ROWS_PT)
    def _writeback():
        pltpu.sync_copy(acc.at[pl.ds(off, ROWS_PT)],
                        out.at[cid, pl.ds(off, ROWS_PT)])


def _mm_body(x_ref, w_ref, o_ref):
    wcat = jnp.concatenate([w_ref[0], w_ref[1]], axis=1)
    res = jnp.dot(x_ref[...], wcat, preferred_element_type=jnp.float32)
    o_ref[0] = res[:, :D]
    o_ref[1] = res[:, D:]


def _mm_batched(x, w):
    k = w.shape[0]
    return pl.pallas_call(
        _mm_body,
        grid=(N // BN, k // 2),
        in_specs=[
            pl.BlockSpec((BN, D), lambda j, r: (j, 0)),
            pl.BlockSpec((2, D, D), lambda j, r: (r, 0, 0)),
        ],
        out_specs=pl.BlockSpec((2, BN, D), lambda j, r: (r, j, 0)),
        out_shape=jax.ShapeDtypeStruct((k, N, D), jnp.float32),
    )(x, w)


def _relu_mm_body(p_ref, w_ref, o_ref):
    f = jnp.maximum(p_ref[0] + p_ref[1], 0.0)
    wcat = jnp.concatenate([w_ref[0], w_ref[1]], axis=1)
    res = jnp.dot(f, wcat, preferred_element_type=jnp.float32)
    o_ref[0] = res[:, :D]
    o_ref[1] = res[:, D:]


def _relu_mm(p, w):
    return pl.pallas_call(
        _relu_mm_body,
        grid=(N // BN, R // 2),
        in_specs=[
            pl.BlockSpec((2, BN, D), lambda j, r: (0, j, 0)),
            pl.BlockSpec((2, D, D), lambda j, r: (r, 0, 0)),
        ],
        out_specs=pl.BlockSpec((2, BN, D), lambda j, r: (r, j, 0)),
        out_shape=jax.ShapeDtypeStruct((R, N, D), jnp.float32),
    )(p, w)


def _att_body(q1_ref, q2_ref, q3_ref, w1_ref, b1_ref, w2_ref, o_ref):
    e1 = q1_ref[0] + q1_ref[1]
    e2 = q2_ref[0] + q2_ref[1]
    e3 = q3_ref[0] + q3_ref[1]
    w1 = w1_ref[...]
    b1 = b1_ref[...]
    w2 = w2_ref[...]

    def score(e):
        h = jnp.tanh(jnp.dot(e, w1, preferred_element_type=jnp.float32) + b1)
        return jnp.sum(h * w2, axis=1, keepdims=True)

    s1, s2, s3 = score(e1), score(e2), score(e3)
    m = jnp.maximum(jnp.maximum(s1, s2), s3)
    a1 = jnp.exp(s1 - m)
    a2 = jnp.exp(s2 - m)
    a3 = jnp.exp(s3 - m)
    denom = a1 + a2 + a3
    o_ref[...] = (a1 * e1 + a2 * e2 + a3 * e3) / denom


def _attention(q1, q2, q3, att_w1, att_b1, att_w2):
    b1 = att_b1.reshape(1, D)
    w2 = att_w2.reshape(1, D)
    return pl.pallas_call(
        _att_body,
        grid=(N // BN,),
        in_specs=[
            pl.BlockSpec((2, BN, D), lambda j: (0, j, 0)),
            pl.BlockSpec((2, BN, D), lambda j: (0, j, 0)),
            pl.BlockSpec((2, BN, D), lambda j: (0, j, 0)),
            pl.BlockSpec((D, D), lambda j: (0, 0)),
            pl.BlockSpec((1, D), lambda j: (0, 0)),
            pl.BlockSpec((1, D), lambda j: (0, 0)),
        ],
        out_specs=pl.BlockSpec((BN, D), lambda j: (j, 0)),
        out_shape=jax.ShapeDtypeStruct((N, D), jnp.float32),
    )(q1, q2, q3, att_w1, b1, w2)


def _edge_arrays(edge_index, rel_type, edge_weight):
    src = edge_index[0]
    dst = edge_index[1]
    gidx = rel_type * N + src
    pad = E_PAD - E
    pad_idx = (jnp.arange(pad, dtype=jnp.int32) * 8) & 16383
    gidx = jnp.concatenate([gidx, pad_idx]).reshape(NW, NCHUNK, CHUNK)
    dst = jnp.pad(dst, (0, pad)).reshape(NW, NCHUNK, CHUNK)
    w = jnp.pad(edge_weight, (0, pad)).reshape(NW, NCHUNK, CHUNK)
    return gidx, dst, w


def kernel(node_features, edge_index1, rel_type1, edge_weight1, edge_index2,
           rel_type2, edge_weight2, edge_index3, rel_type3, edge_weight3,
           rel_emb_v1_l1, rel_emb_v1_l2, rel_emb_v2_l1, rel_emb_v2_l2,
           rel_emb_v3_l1, rel_emb_v3_l2, att_w1, att_b1, att_w2, att_b2):
    del att_b2
    zeros = jnp.zeros((N, D), jnp.float32)
    edges = [
        _edge_arrays(edge_index1, rel_type1, edge_weight1),
        _edge_arrays(edge_index2, rel_type2, edge_weight2),
        _edge_arrays(edge_index3, rel_type3, edge_weight3),
    ]
    w_l1 = jnp.concatenate([rel_emb_v1_l1, rel_emb_v2_l1, rel_emb_v3_l1], 0)
    w_l2 = [rel_emb_v1_l2, rel_emb_v2_l2, rel_emb_v3_l2]

    y1 = _mm_batched(node_features, w_l1)

    p2 = []
    for v in range(NV):
        gidx, dst, w = edges[v]
        table1 = y1[R * v:R * v + R].reshape(R * N, D)
        p1 = _edge_pass(table1, gidx, dst, w, zeros)
        y2 = _relu_mm(p1, w_l2[v]).reshape(R * N, D)
        p2.append(_edge_pass(y2, gidx, dst, w, zeros))

    return _attention(p2[0], p2[1], p2[2], att_w1, att_b1, att_w2)


# speedup vs baseline: 3.6002x; 1.0098x over previous
import functools

import jax
import jax.numpy as jnp
import numpy as np
from jax import lax
from jax.experimental import pallas as pl
from jax.experimental.pallas import tpu as pltpu
from jax.experimental.pallas import tpu_sc as plsc

N = 10000
E = 160000
D = 128
R = 4
NV = 3

NW = 32
CHUNK = 128
NCHUNK = 40
EPW = NCHUNK * CHUNK
E_PAD = NW * EPW
ROWS_PT = 1000

BN = 1000


_sc_mesh = plsc.VectorSubcoreMesh(core_axis_name="c", subcore_axis_name="s")


@functools.partial(
    pl.kernel,
    out_type=jax.ShapeDtypeStruct((2, N, D), jnp.float32),
    mesh=_sc_mesh,
    scratch_types=[
        pltpu.VMEM((NCHUNK, CHUNK), jnp.int32),
        pltpu.VMEM((NCHUNK, CHUNK), jnp.int32),
        pltpu.VMEM((NCHUNK, CHUNK), jnp.float32),
        [pltpu.VMEM((CHUNK, D), jnp.float32)] * 2,
        pltpu.VMEM_SHARED((N, D), jnp.float32),
        [pltpu.SemaphoreType.DMA] * 4,
    ],
)
def _edge_pass(table, gidx, dst, w, zeros, out, gidx_v, dst_v, w_v, rows,
               acc, sems):
    cid = lax.axis_index("c")
    sid = lax.axis_index("s")
    wid = cid * 16 + sid

    off = pl.multiple_of(sid * 640, 8)

    pltpu.sync_copy(gidx.at[wid], gidx_v)
    pltpu.sync_copy(dst.at[wid], dst_v)
    pltpu.sync_copy(w.at[wid], w_v)

    def scale_scatter(buf, c):
        def grp_body(g, carry2):
            wrow = w_v[c, pl.ds(g * 16, 16)]
            for k in range(16):
                wvec = jnp.full((16,), wrow[k], dtype=jnp.float32)
                i = g * 16 + k
                for l in range(8):
                    sl = pl.ds(l * 16, 16)
                    buf[i, sl] = buf[i, sl] * wvec
            return carry2

        lax.fori_loop(0, CHUNK // 16, grp_body, 0)
        pltpu.sync_copy(buf, acc.at[dst_v.at[c]], add=True)

    def issue(c, b):
        pltpu.async_copy(table.at[gidx_v.at[c]], rows[b], sems[b])

    def drain(c, b):
        pltpu.make_async_copy(table.at[gidx_v.at[c]], rows[b], sems[b]).wait()

    NB = 2
    for b in range(NB):
        issue(b, b)

    @pl.when(sid < 15)
    def _init():
        pltpu.sync_copy(zeros.at[pl.ds(off, 640)], acc.at[pl.ds(off, 640)])

    @pl.when(sid == 15)
    def _init_last():
        pltpu.sync_copy(zeros.at[pl.ds(9600, 400)], acc.at[pl.ds(9600, 400)])

    plsc.subcore_barrier()

    def ring_body(t, carry):
        for b in range(NB):
            c = t * NB + b
            drain(c, b)
            scale_scatter(rows[b], c)
            issue(c + NB, b)
        return carry

    lax.fori_loop(0, NCHUNK // NB - 1, ring_body, 0)
    for b in range(NB):
        c = NCHUNK - NB + b
        drain(c, b)
        scale_scatter(rows[b], c)
    plsc.subcore_barrier()

    @pl.when(sid < 15)
    def _writeback():
        pltpu.sync_copy(acc.at[pl.ds(off, 640)],
                        out.at[cid, pl.ds(off, 640)])

    @pl.when(sid == 15)
    def _writeback_last():
        pltpu.sync_copy(acc.at[pl.ds(9600, 400)],
                        out.at[cid, pl.ds(9600, 400)])


def _mm_body(x_ref, w_ref, o_ref):
    wcat = jnp.concatenate([w_ref[0], w_ref[1]], axis=1)
    res = jnp.dot(x_ref[...], wcat, preferred_element_type=jnp.float32)
    o_ref[0] = res[:, :D]
    o_ref[1] = res[:, D:]


def _mm_batched(x, w):
    k = w.shape[0]
    return pl.pallas_call(
        _mm_body,
        grid=(N // BN, k // 2),
        in_specs=[
            pl.BlockSpec((BN, D), lambda j, r: (j, 0)),
            pl.BlockSpec((2, D, D), lambda j, r: (r, 0, 0)),
        ],
        out_specs=pl.BlockSpec((2, BN, D), lambda j, r: (r, j, 0)),
        out_shape=jax.ShapeDtypeStruct((k, N, D), jnp.float32),
    )(x, w)


def _relu_mm_body(p_ref, w_ref, o_ref):
    f = jnp.maximum(p_ref[0] + p_ref[1], 0.0)
    wcat = jnp.concatenate([w_ref[0], w_ref[1]], axis=1)
    res = jnp.dot(f, wcat, preferred_element_type=jnp.float32)
    o_ref[0] = res[:, :D]
    o_ref[1] = res[:, D:]


def _relu_mm(p, w):
    return pl.pallas_call(
        _relu_mm_body,
        grid=(N // BN, R // 2),
        in_specs=[
            pl.BlockSpec((2, BN, D), lambda j, r: (0, j, 0)),
            pl.BlockSpec((2, D, D), lambda j, r: (r, 0, 0)),
        ],
        out_specs=pl.BlockSpec((2, BN, D), lambda j, r: (r, j, 0)),
        out_shape=jax.ShapeDtypeStruct((R, N, D), jnp.float32),
    )(p, w)


def _att_body(q1_ref, q2_ref, q3_ref, w1_ref, b1_ref, w2_ref, o_ref):
    e1 = q1_ref[0] + q1_ref[1]
    e2 = q2_ref[0] + q2_ref[1]
    e3 = q3_ref[0] + q3_ref[1]
    w1 = w1_ref[...]
    b1 = b1_ref[...]
    w2 = w2_ref[...]

    def score(e):
        h = jnp.tanh(jnp.dot(e, w1, preferred_element_type=jnp.float32) + b1)
        return jnp.sum(h * w2, axis=1, keepdims=True)

    s1, s2, s3 = score(e1), score(e2), score(e3)
    m = jnp.maximum(jnp.maximum(s1, s2), s3)
    a1 = jnp.exp(s1 - m)
    a2 = jnp.exp(s2 - m)
    a3 = jnp.exp(s3 - m)
    denom = a1 + a2 + a3
    o_ref[...] = (a1 * e1 + a2 * e2 + a3 * e3) / denom


def _attention(q1, q2, q3, att_w1, att_b1, att_w2):
    b1 = att_b1.reshape(1, D)
    w2 = att_w2.reshape(1, D)
    return pl.pallas_call(
        _att_body,
        grid=(N // BN,),
        in_specs=[
            pl.BlockSpec((2, BN, D), lambda j: (0, j, 0)),
            pl.BlockSpec((2, BN, D), lambda j: (0, j, 0)),
            pl.BlockSpec((2, BN, D), lambda j: (0, j, 0)),
            pl.BlockSpec((D, D), lambda j: (0, 0)),
            pl.BlockSpec((1, D), lambda j: (0, 0)),
            pl.BlockSpec((1, D), lambda j: (0, 0)),
        ],
        out_specs=pl.BlockSpec((BN, D), lambda j: (j, 0)),
        out_shape=jax.ShapeDtypeStruct((N, D), jnp.float32),
    )(q1, q2, q3, att_w1, b1, w2)


def _edge_arrays(edge_index, rel_type, edge_weight):
    src = edge_index[0]
    dst = edge_index[1]
    gidx = rel_type * N + src
    pad = E_PAD - E
    pad_idx = (jnp.arange(pad, dtype=jnp.int32) * 8) & 16383
    gidx = jnp.concatenate([gidx, pad_idx]).reshape(NW, NCHUNK, CHUNK)
    dst = jnp.pad(dst, (0, pad)).reshape(NW, NCHUNK, CHUNK)
    w = jnp.pad(edge_weight, (0, pad)).reshape(NW, NCHUNK, CHUNK)
    return gidx, dst, w


def kernel(node_features, edge_index1, rel_type1, edge_weight1, edge_index2,
           rel_type2, edge_weight2, edge_index3, rel_type3, edge_weight3,
           rel_emb_v1_l1, rel_emb_v1_l2, rel_emb_v2_l1, rel_emb_v2_l2,
           rel_emb_v3_l1, rel_emb_v3_l2, att_w1, att_b1, att_w2, att_b2):
    del att_b2
    zeros = jnp.zeros((N, D), jnp.float32)
    edges = [
        _edge_arrays(edge_index1, rel_type1, edge_weight1),
        _edge_arrays(edge_index2, rel_type2, edge_weight2),
        _edge_arrays(edge_index3, rel_type3, edge_weight3),
    ]
    w_l1 = jnp.concatenate([rel_emb_v1_l1, rel_emb_v2_l1, rel_emb_v3_l1], 0)
    w_l2 = [rel_emb_v1_l2, rel_emb_v2_l2, rel_emb_v3_l2]

    y1 = _mm_batched(node_features, w_l1)

    p2 = []
    for v in range(NV):
        gidx, dst, w = edges[v]
        table1 = y1[R * v:R * v + R].reshape(R * N, D)
        p1 = _edge_pass(table1, gidx, dst, w, zeros)
        y2 = _relu_mm(p1, w_l2[v]).reshape(R * N, D)
        p2.append(_edge_pass(y2, gidx, dst, w, zeros))

    return _attention(p2[0], p2[1], p2[2], att_w1, att_b1, att_w2)


# speedup vs baseline: 3.9937x; 1.1093x over previous
import functools

import jax
import jax.numpy as jnp
import numpy as np
from jax import lax
from jax.experimental import pallas as pl
from jax.experimental.pallas import tpu as pltpu
from jax.experimental.pallas import tpu_sc as plsc

N = 10000
E = 160000
D = 128
R = 4
NV = 3

NW = 32
CHUNK = 128
NCHUNK = 40
EPW = NCHUNK * CHUNK
E_PAD = NW * EPW
ROWS_PT = 1000

BN = 1000


_sc_mesh = plsc.VectorSubcoreMesh(core_axis_name="c", subcore_axis_name="s")


@functools.partial(
    pl.kernel,
    out_type=jax.ShapeDtypeStruct((2, N, D), jnp.float32),
    mesh=_sc_mesh,
    scratch_types=[
        pltpu.VMEM((NCHUNK, CHUNK), jnp.int32),
        pltpu.VMEM((NCHUNK, CHUNK), jnp.int32),
        pltpu.VMEM((NCHUNK, CHUNK), jnp.float32),
        [pltpu.VMEM((CHUNK, D), jnp.float32)] * 2,
        pltpu.VMEM_SHARED((N, D), jnp.float32),
        [pltpu.SemaphoreType.DMA] * 4,
    ],
)
def _edge_pass(table, gidx, dst, w, zeros, out, gidx_v, dst_v, w_v, rows,
               acc, sems):
    cid = lax.axis_index("c")
    sid = lax.axis_index("s")
    wid = cid * 16 + sid

    off = pl.multiple_of(sid * 640, 8)

    pltpu.sync_copy(gidx.at[wid], gidx_v)
    pltpu.sync_copy(dst.at[wid], dst_v)
    pltpu.sync_copy(w.at[wid], w_v)

    def scale_scatter(buf, c):
        def grp_body(g, carry2):
            wrow = w_v[c, pl.ds(g * 16, 16)]
            for k in range(16):
                wvec = jnp.full((16,), wrow[k], dtype=jnp.float32)
                i = g * 16 + k
                for l in range(8):
                    sl = pl.ds(l * 16, 16)
                    buf[i, sl] = buf[i, sl] * wvec
            return carry2

        lax.fori_loop(0, CHUNK // 16, grp_body, 0)
        pltpu.sync_copy(buf, acc.at[dst_v.at[c]], add=True)

    def issue(c, b):
        pltpu.async_copy(table.at[gidx_v.at[c]], rows[b], sems[b])

    def drain(c, b):
        pltpu.make_async_copy(table.at[gidx_v.at[c]], rows[b], sems[b]).wait()

    NB = 2
    for b in range(NB):
        issue(b, b)

    @pl.when(sid < 15)
    def _init():
        pltpu.sync_copy(zeros.at[pl.ds(off, 640)], acc.at[pl.ds(off, 640)])

    @pl.when(sid == 15)
    def _init_last():
        pltpu.sync_copy(zeros.at[pl.ds(9600, 400)], acc.at[pl.ds(9600, 400)])

    plsc.subcore_barrier()

    def ring_body(t, carry):
        for b in range(NB):
            c = t * NB + b
            drain(c, b)
            scale_scatter(rows[b], c)
            issue(c + NB, b)
        return carry

    lax.fori_loop(0, NCHUNK // NB - 1, ring_body, 0)
    for b in range(NB):
        c = NCHUNK - NB + b
        drain(c, b)
        scale_scatter(rows[b], c)
    plsc.subcore_barrier()

    @pl.when(sid < 15)
    def _writeback():
        pltpu.sync_copy(acc.at[pl.ds(off, 640)],
                        out.at[cid, pl.ds(off, 640)])

    @pl.when(sid == 15)
    def _writeback_last():
        pltpu.sync_copy(acc.at[pl.ds(9600, 400)],
                        out.at[cid, pl.ds(9600, 400)])


def _mm_body(x_ref, w_ref, o_ref):
    wcat = jnp.concatenate([w_ref[0], w_ref[1]], axis=1)
    res = jnp.dot(x_ref[...], wcat, preferred_element_type=jnp.float32)
    o_ref[0] = res[:, :D]
    o_ref[1] = res[:, D:]


def _mm_batched(x, w):
    k = w.shape[0]
    return pl.pallas_call(
        _mm_body,
        grid=(N // BN, k // 2),
        in_specs=[
            pl.BlockSpec((BN, D), lambda j, r: (j, 0)),
            pl.BlockSpec((2, D, D), lambda j, r: (r, 0, 0)),
        ],
        out_specs=pl.BlockSpec((2, BN, D), lambda j, r: (r, j, 0)),
        out_shape=jax.ShapeDtypeStruct((k, N, D), jnp.float32),
    )(x, w)


def _relu_mm_body(p_ref, w_ref, o_ref):
    f = jnp.maximum(p_ref[0] + p_ref[1], 0.0)
    wcat = jnp.concatenate([w_ref[0], w_ref[1]], axis=1)
    res = jnp.dot(f, wcat, preferred_element_type=jnp.float32)
    o_ref[0] = res[:, :D]
    o_ref[1] = res[:, D:]


def _relu_mm(p, w):
    return pl.pallas_call(
        _relu_mm_body,
        grid=(N // BN, R // 2),
        in_specs=[
            pl.BlockSpec((2, BN, D), lambda j, r: (0, j, 0)),
            pl.BlockSpec((2, D, D), lambda j, r: (r, 0, 0)),
        ],
        out_specs=pl.BlockSpec((2, BN, D), lambda j, r: (r, j, 0)),
        out_shape=jax.ShapeDtypeStruct((R, N, D), jnp.float32),
    )(p, w)


def _att_body(q1_ref, q2_ref, q3_ref, w1_ref, b1_ref, w2_ref, o_ref):
    e1 = q1_ref[0] + q1_ref[1]
    e2 = q2_ref[0] + q2_ref[1]
    e3 = q3_ref[0] + q3_ref[1]
    w1 = w1_ref[...]
    b1 = b1_ref[...]
    w2 = w2_ref[...]

    def score(e):
        h = jnp.tanh(jnp.dot(e, w1, preferred_element_type=jnp.float32) + b1)
        return jnp.sum(h * w2, axis=1, keepdims=True)

    s1, s2, s3 = score(e1), score(e2), score(e3)
    m = jnp.maximum(jnp.maximum(s1, s2), s3)
    a1 = jnp.exp(s1 - m)
    a2 = jnp.exp(s2 - m)
    a3 = jnp.exp(s3 - m)
    denom = a1 + a2 + a3
    o_ref[...] = (a1 * e1 + a2 * e2 + a3 * e3) / denom


def _attention(q1, q2, q3, att_w1, att_b1, att_w2):
    b1 = att_b1.reshape(1, D)
    w2 = att_w2.reshape(1, D)
    return pl.pallas_call(
        _att_body,
        grid=(N // BN,),
        in_specs=[
            pl.BlockSpec((2, BN, D), lambda j: (0, j, 0)),
            pl.BlockSpec((2, BN, D), lambda j: (0, j, 0)),
            pl.BlockSpec((2, BN, D), lambda j: (0, j, 0)),
            pl.BlockSpec((D, D), lambda j: (0, 0)),
            pl.BlockSpec((1, D), lambda j: (0, 0)),
            pl.BlockSpec((1, D), lambda j: (0, 0)),
        ],
        out_specs=pl.BlockSpec((BN, D), lambda j: (j, 0)),
        out_shape=jax.ShapeDtypeStruct((N, D), jnp.float32),
    )(q1, q2, q3, att_w1, b1, w2)


def _edge_arrays(edge_index, rel_type, edge_weight):
    src = edge_index[0]
    dst = edge_index[1]
    gidx = rel_type * N + src
    pad = E_PAD - E
    pad_idx = (jnp.arange(pad, dtype=jnp.int32) * 8) & 16383
    gidx = jnp.concatenate([gidx, pad_idx]).reshape(NW, NCHUNK, CHUNK)
    dst = jnp.pad(dst, (0, pad)).reshape(NW, NCHUNK, CHUNK)
    w = jnp.pad(edge_weight, (0, pad)).reshape(NW, NCHUNK, CHUNK)
    return gidx, dst, w


def kernel(node_features, edge_index1, rel_type1, edge_weight1, edge_index2,
           rel_type2, edge_weight2, edge_index3, rel_type3, edge_weight3,
           rel_emb_v1_l1, rel_emb_v1_l2, rel_emb_v2_l1, rel_emb_v2_l2,
           rel_emb_v3_l1, rel_emb_v3_l2, att_w1, att_b1, att_w2, att_b2):
    del att_b2
    zeros = jnp.zeros((N, D), jnp.float32)
    edges = [
        _edge_arrays(edge_index1, rel_type1, edge_weight1),
        _edge_arrays(edge_index2, rel_type2, edge_weight2),
        _edge_arrays(edge_index3, rel_type3, edge_weight3),
    ]
    w_l1 = [rel_emb_v1_l1, rel_emb_v2_l1, rel_emb_v3_l1]
    w_l2 = [rel_emb_v1_l2, rel_emb_v2_l2, rel_emb_v3_l2]

    y1 = [_mm_batched(node_features, w_l1[v]).reshape(R * N, D)
          for v in range(NV)]
    p1 = [_edge_pass(y1[v], *edges[v], zeros) for v in range(NV)]
    y2 = [_relu_mm(p1[v], w_l2[v]).reshape(R * N, D) for v in range(NV)]
    p2 = [_edge_pass(y2[v], *edges[v], zeros) for v in range(NV)]

    return _attention(p2[0], p2[1], p2[2], att_w1, att_b1, att_w2)
